# Initial kernel scaffold; baseline (speedup 1.0000x reference)
import functools

import jax
import jax.numpy as jnp
from jax import lax
from jax.experimental import pallas as pl
from jax.experimental.pallas import tpu as pltpu
from jax.experimental.pallas import tpu_sc as plsc

N = 10000
E = 160000
D = 256
H = 256
C = 40
C_PAD = 48
HALF = 128

NC = 2
NS = 16
CHUNK = 128
NCHUNK = E // CHUNK
ROWS_PER_TILE = N // NS

BLK = 400
GRID = N // BLK

_mesh = plsc.VectorSubcoreMesh(
    core_axis_name="c", subcore_axis_name="s", num_cores=NC, num_subcores=NS
)


@functools.partial(
    pl.kernel,
    out_type=jax.ShapeDtypeStruct((NC, N, 16), jnp.float32),
    mesh=_mesh,
    scratch_types=[
        pltpu.VMEM((CHUNK,), jnp.int32),
        pltpu.VMEM((CHUNK, 16), jnp.float32),
        pltpu.VMEM_SHARED((N, 16), jnp.float32),
    ],
)
def _sc_degree(dst_hbm, ones_hbm, zero_hbm, out_hbm, idx_v, ones_v, acc):
    c = lax.axis_index("c")
    s = lax.axis_index("s")
    pltpu.sync_copy(ones_hbm, ones_v)
    pltpu.sync_copy(zero_hbm, acc.at[pl.ds(s * ROWS_PER_TILE, ROWS_PER_TILE)])
    plsc.subcore_barrier()

    nj = 39 + jnp.where(s < 1, 1, 0)

    def body(j, carry):
        ch = c * (NCHUNK // NC) + s + j * NS
        pltpu.sync_copy(dst_hbm.at[pl.ds(ch * CHUNK, CHUNK)], idx_v)
        pltpu.sync_copy(ones_v, acc.at[idx_v], add=True)
        return carry

    lax.fori_loop(0, nj, body, 0)
    plsc.subcore_barrier()
    pltpu.sync_copy(
        acc.at[pl.ds(s * ROWS_PER_TILE, ROWS_PER_TILE)],
        out_hbm.at[c, pl.ds(s * ROWS_PER_TILE, ROWS_PER_TILE)],
    )


@functools.partial(
    pl.kernel,
    out_type=jax.ShapeDtypeStruct((NC, N, HALF), jnp.float32),
    mesh=_mesh,
    scratch_types=[
        pltpu.VMEM((CHUNK,), jnp.int32),
        pltpu.VMEM((CHUNK,), jnp.int32),
        pltpu.VMEM((CHUNK, HALF), jnp.float32),
        pltpu.VMEM_SHARED((N, HALF), jnp.float32),
        pltpu.SemaphoreType.DMA,
    ],
)
def _sc_prop1(hp_hbm, src_hbm, dst_hbm, zero_hbm, out_hbm,
              src_v, dst_v, rows_v, acc, sem):
    c = lax.axis_index("c")
    s = lax.axis_index("s")
    pltpu.sync_copy(zero_hbm, acc.at[pl.ds(s * ROWS_PER_TILE, ROWS_PER_TILE)])
    plsc.subcore_barrier()

    off = c * N
    nj = 78 + jnp.where(s < 2, 1, 0)

    def body(j, carry):
        ch = s + j * NS
        base = ch * CHUNK
        pltpu.sync_copy(src_hbm.at[pl.ds(base, CHUNK)], src_v)
        pltpu.sync_copy(dst_hbm.at[pl.ds(base, CHUNK)], dst_v)
        for t in range(CHUNK // 16):
            sl = pl.ds(t * 16, 16)
            src_v[sl] = src_v[sl] + off
        pltpu.async_copy(hp_hbm.at[src_v], rows_v, sem).wait()
        pltpu.sync_copy(rows_v, acc.at[dst_v], add=True)
        return carry

    lax.fori_loop(0, nj, body, 0)
    plsc.subcore_barrier()
    pltpu.sync_copy(
        acc.at[pl.ds(s * ROWS_PER_TILE, ROWS_PER_TILE)],
        out_hbm.at[c, pl.ds(s * ROWS_PER_TILE, ROWS_PER_TILE)],
    )


@functools.partial(
    pl.kernel,
    out_type=jax.ShapeDtypeStruct((NC, N, C_PAD), jnp.float32),
    mesh=_mesh,
    scratch_types=[
        pltpu.VMEM((CHUNK,), jnp.int32),
        pltpu.VMEM((CHUNK,), jnp.int32),
        pltpu.VMEM((CHUNK, C_PAD), jnp.float32),
        pltpu.VMEM_SHARED((N, C_PAD), jnp.float32),
        pltpu.SemaphoreType.DMA,
    ],
)
def _sc_prop2(h2p_hbm, src_hbm, dst_hbm, zero_hbm, out_hbm,
              src_v, dst_v, rows_v, acc, sem):
    c = lax.axis_index("c")
    s = lax.axis_index("s")
    pltpu.sync_copy(zero_hbm, acc.at[pl.ds(s * ROWS_PER_TILE, ROWS_PER_TILE)])
    plsc.subcore_barrier()

    nj = 39 + jnp.where(s < 1, 1, 0)

    def body(j, carry):
        ch = c * (NCHUNK // NC) + s + j * NS
        base = ch * CHUNK
        pltpu.sync_copy(src_hbm.at[pl.ds(base, CHUNK)], src_v)
        pltpu.sync_copy(dst_hbm.at[pl.ds(base, CHUNK)], dst_v)
        pltpu.async_copy(h2p_hbm.at[src_v], rows_v, sem).wait()
        pltpu.sync_copy(rows_v, acc.at[dst_v], add=True)
        return carry

    lax.fori_loop(0, nj, body, 0)
    plsc.subcore_barrier()
    pltpu.sync_copy(
        acc.at[pl.ds(s * ROWS_PER_TILE, ROWS_PER_TILE)],
        out_hbm.at[c, pl.ds(s * ROWS_PER_TILE, ROWS_PER_TILE)],
    )


def _dinv_body(degp_ref, dinv_ref):
    deg = degp_ref[0, :, 0] + degp_ref[1, :, 0] + 1.0
    dinv_ref[:, 0] = 1.0 / jnp.sqrt(deg)


def _tc_dinv(degp):
    return pl.pallas_call(
        _dinv_body,
        out_shape=jax.ShapeDtypeStruct((N, 1), jnp.float32),
    )(degp)


def _l1_body(x_ref, w1_ref, dinv_ref, hp_ref):
    h = jnp.dot(x_ref[...], w1_ref[...], preferred_element_type=jnp.float32)
    hp = dinv_ref[...] * h
    hp_ref[0] = hp[:, :HALF]
    hp_ref[1] = hp[:, HALF:]


def _tc_l1(x, w1, dinv):
    return pl.pallas_call(
        _l1_body,
        grid=(GRID,),
        in_specs=[
            pl.BlockSpec((BLK, D), lambda i: (i, 0)),
            pl.BlockSpec((D, H), lambda i: (0, 0)),
            pl.BlockSpec((BLK, 1), lambda i: (i, 0)),
        ],
        out_specs=pl.BlockSpec((NC, BLK, HALF), lambda i: (0, i, 0)),
        out_shape=jax.ShapeDtypeStruct((NC, N, HALF), jnp.float32),
    )(x, w1, dinv)


def _l2_body(acc_ref, hp_ref, dinv_ref, b1_ref, w2_ref, h2p_ref):
    a = jnp.concatenate([acc_ref[0], acc_ref[1]], axis=-1)
    hp = jnp.concatenate([hp_ref[0], hp_ref[1]], axis=-1)
    dinv = dinv_ref[...]
    h1 = jnp.maximum(dinv * (a + hp) + b1_ref[...], 0.0)
    h2p_ref[...] = dinv * jnp.dot(h1, w2_ref[...],
                                  preferred_element_type=jnp.float32)


def _tc_l2(acc1, hp, dinv, b1, w2p):
    return pl.pallas_call(
        _l2_body,
        grid=(GRID,),
        in_specs=[
            pl.BlockSpec((NC, BLK, HALF), lambda i: (0, i, 0)),
            pl.BlockSpec((NC, BLK, HALF), lambda i: (0, i, 0)),
            pl.BlockSpec((BLK, 1), lambda i: (i, 0)),
            pl.BlockSpec((1, H), lambda i: (0, 0)),
            pl.BlockSpec((H, C_PAD), lambda i: (0, 0)),
        ],
        out_specs=pl.BlockSpec((BLK, C_PAD), lambda i: (i, 0)),
        out_shape=jax.ShapeDtypeStruct((N, C_PAD), jnp.float32),
    )(acc1, hp, dinv, b1, w2p)


def _final_body(acc_ref, h2p_ref, dinv_ref, b2_ref, out_ref):
    a = acc_ref[0] + acc_ref[1] + h2p_ref[...]
    o = dinv_ref[...] * a + b2_ref[...]
    o = o[:, :C]
    m = jnp.max(o, axis=1, keepdims=True)
    z = o - m
    lse = jnp.log(jnp.sum(jnp.exp(z), axis=1, keepdims=True))
    out_ref[...] = z - lse


def _tc_final(acc2, h2p, dinv, b2):
    return pl.pallas_call(
        _final_body,
        grid=(GRID,),
        in_specs=[
            pl.BlockSpec((NC, BLK, C_PAD), lambda i: (0, i, 0)),
            pl.BlockSpec((BLK, C_PAD), lambda i: (i, 0)),
            pl.BlockSpec((BLK, 1), lambda i: (i, 0)),
            pl.BlockSpec((1, C), lambda i: (0, 0)),
        ],
        out_specs=pl.BlockSpec((BLK, C), lambda i: (i, 0)),
        out_shape=jax.ShapeDtypeStruct((N, C), jnp.float32),
    )(acc2, h2p, dinv, b2)


@jax.jit
def kernel(x, edge_index, W1, b1, W2, b2):
    src = edge_index[0]
    dst = edge_index[1]

    ones16 = jnp.ones((CHUNK, 16), jnp.float32)
    zero16 = jnp.zeros((ROWS_PER_TILE, 16), jnp.float32)
    zero_half = jnp.zeros((ROWS_PER_TILE, HALF), jnp.float32)
    zero_cp = jnp.zeros((ROWS_PER_TILE, C_PAD), jnp.float32)
    w2p = jnp.zeros((H, C_PAD), jnp.float32).at[:, :C].set(W2)

    degp = _sc_degree(dst, ones16, zero16)
    dinv = _tc_dinv(degp)
    hp = _tc_l1(x, W1, dinv)
    acc1 = _sc_prop1(hp.reshape(NC * N, HALF), src, dst, zero_half)
    h2p = _tc_l2(acc1, hp, dinv, b1.reshape(1, H), w2p)
    acc2 = _sc_prop2(h2p, src, dst, zero_cp)
    return _tc_final(acc2, h2p, dinv, b2.reshape(1, C))


# speedup vs baseline: 9.7406x; 9.7406x over previous
import functools

import jax
import jax.numpy as jnp
from jax import lax
from jax.experimental import pallas as pl
from jax.experimental.pallas import tpu as pltpu
from jax.experimental.pallas import tpu_sc as plsc

N = 10000
E = 160000
D = 256
H = 256
C = 40
C_PAD = 128
HALF = 128

NC = 2
NS = 16
CHUNK = 128
NCHUNK = E // CHUNK
R0 = 624
RL = N - (NS - 1) * R0

BLK = 400
GRID = N // BLK

_mesh = plsc.VectorSubcoreMesh(
    core_axis_name="c", subcore_axis_name="s", num_cores=NC, num_subcores=NS
)


def _zero_fill(zero_hbm, acc, s):
    @pl.when(s < NS - 1)
    def _():
        pltpu.sync_copy(zero_hbm.at[pl.ds(0, R0)], acc.at[pl.ds(s * R0, R0)])

    @pl.when(s == NS - 1)
    def _():
        pltpu.sync_copy(zero_hbm, acc.at[pl.ds((NS - 1) * R0, RL)])


def _writeback(acc, out_hbm, c, s):
    @pl.when(s < NS - 1)
    def _():
        pltpu.sync_copy(acc.at[pl.ds(s * R0, R0)],
                        out_hbm.at[c, pl.ds(s * R0, R0)])

    @pl.when(s == NS - 1)
    def _():
        pltpu.sync_copy(acc.at[pl.ds((NS - 1) * R0, RL)],
                        out_hbm.at[c, pl.ds((NS - 1) * R0, RL)])

@functools.partial(
    pl.kernel,
    out_type=jax.ShapeDtypeStruct((NC, N, 128), jnp.float32),
    mesh=_mesh,
    scratch_types=[
        pltpu.VMEM((CHUNK,), jnp.int32),
        pltpu.VMEM((CHUNK, 128), jnp.float32),
        pltpu.VMEM_SHARED((N, 128), jnp.float32),
    ],
)
def _sc_degree(dst_hbm, ones_hbm, zero_hbm, out_hbm, idx_v, ones_v, acc):
    c = lax.axis_index("c")
    s = lax.axis_index("s")
    pltpu.sync_copy(ones_hbm, ones_v)
    _zero_fill(zero_hbm, acc, s)
    plsc.subcore_barrier()

    nj = 39 + jnp.where(s < 1, 1, 0)

    def body(j, carry):
        ch = c * (NCHUNK // NC) + s + j * NS
        pltpu.sync_copy(dst_hbm.at[pl.ds(ch * CHUNK, CHUNK)], idx_v)
        pltpu.sync_copy(ones_v, acc.at[idx_v], add=True)
        return carry

    lax.fori_loop(0, nj, body, 0)
    plsc.subcore_barrier()
    _writeback(acc, out_hbm, c, s)


@functools.partial(
    pl.kernel,
    out_type=jax.ShapeDtypeStruct((NC, N, HALF), jnp.float32),
    mesh=_mesh,
    scratch_types=[
        pltpu.VMEM((CHUNK,), jnp.int32),
        pltpu.VMEM((CHUNK,), jnp.int32),
        pltpu.VMEM((CHUNK, HALF), jnp.float32),
        pltpu.VMEM_SHARED((N, HALF), jnp.float32),
        pltpu.SemaphoreType.DMA,
    ],
)
def _sc_prop1(hp_hbm, src_hbm, dst_hbm, zero_hbm, out_hbm,
              src_v, dst_v, rows_v, acc, sem):
    c = lax.axis_index("c")
    s = lax.axis_index("s")
    _zero_fill(zero_hbm, acc, s)
    plsc.subcore_barrier()

    off = c * N
    nj = 78 + jnp.where(s < 2, 1, 0)

    def body(j, carry):
        ch = s + j * NS
        base = ch * CHUNK
        pltpu.sync_copy(src_hbm.at[pl.ds(base, CHUNK)], src_v)
        pltpu.sync_copy(dst_hbm.at[pl.ds(base, CHUNK)], dst_v)
        for t in range(CHUNK // 16):
            sl = pl.ds(t * 16, 16)
            src_v[sl] = src_v[sl] + off
        pltpu.async_copy(hp_hbm.at[src_v], rows_v, sem).wait()
        pltpu.sync_copy(rows_v, acc.at[dst_v], add=True)
        return carry

    lax.fori_loop(0, nj, body, 0)
    plsc.subcore_barrier()
    _writeback(acc, out_hbm, c, s)


@functools.partial(
    pl.kernel,
    out_type=jax.ShapeDtypeStruct((NC, N, C_PAD), jnp.float32),
    mesh=_mesh,
    scratch_types=[
        pltpu.VMEM((CHUNK,), jnp.int32),
        pltpu.VMEM((CHUNK,), jnp.int32),
        pltpu.VMEM((CHUNK, C_PAD), jnp.float32),
        pltpu.VMEM_SHARED((N, C_PAD), jnp.float32),
        pltpu.SemaphoreType.DMA,
    ],
)
def _sc_prop2(h2p_hbm, src_hbm, dst_hbm, zero_hbm, out_hbm,
              src_v, dst_v, rows_v, acc, sem):
    c = lax.axis_index("c")
    s = lax.axis_index("s")
    _zero_fill(zero_hbm, acc, s)
    plsc.subcore_barrier()

    nj = 39 + jnp.where(s < 1, 1, 0)

    def body(j, carry):
        ch = c * (NCHUNK // NC) + s + j * NS
        base = ch * CHUNK
        pltpu.sync_copy(src_hbm.at[pl.ds(base, CHUNK)], src_v)
        pltpu.sync_copy(dst_hbm.at[pl.ds(base, CHUNK)], dst_v)
        pltpu.async_copy(h2p_hbm.at[src_v], rows_v, sem).wait()
        pltpu.sync_copy(rows_v, acc.at[dst_v], add=True)
        return carry

    lax.fori_loop(0, nj, body, 0)
    plsc.subcore_barrier()
    _writeback(acc, out_hbm, c, s)


def _dinv_body(degp_ref, dinv_ref):
    deg = degp_ref[0, :, 0] + degp_ref[1, :, 0] + 1.0
    dinv_ref[:, 0] = 1.0 / jnp.sqrt(deg)


def _tc_dinv(degp):
    return pl.pallas_call(
        _dinv_body,
        out_shape=jax.ShapeDtypeStruct((N, 1), jnp.float32),
    )(degp)


def _l1_body(x_ref, w1_ref, dinv_ref, hp_ref):
    h = jnp.dot(x_ref[...], w1_ref[...], preferred_element_type=jnp.float32)
    hp = dinv_ref[...] * h
    hp_ref[0] = hp[:, :HALF]
    hp_ref[1] = hp[:, HALF:]


def _tc_l1(x, w1, dinv):
    return pl.pallas_call(
        _l1_body,
        grid=(GRID,),
        in_specs=[
            pl.BlockSpec((BLK, D), lambda i: (i, 0)),
            pl.BlockSpec((D, H), lambda i: (0, 0)),
            pl.BlockSpec((BLK, 1), lambda i: (i, 0)),
        ],
        out_specs=pl.BlockSpec((NC, BLK, HALF), lambda i: (0, i, 0)),
        out_shape=jax.ShapeDtypeStruct((NC, N, HALF), jnp.float32),
    )(x, w1, dinv)


def _l2_body(acc_ref, hp_ref, dinv_ref, b1_ref, w2_ref, h2p_ref):
    a = jnp.concatenate([acc_ref[0], acc_ref[1]], axis=-1)
    hp = jnp.concatenate([hp_ref[0], hp_ref[1]], axis=-1)
    dinv = dinv_ref[...]
    h1 = jnp.maximum(dinv * (a + hp) + b1_ref[...], 0.0)
    h2p_ref[...] = dinv * jnp.dot(h1, w2_ref[...],
                                  preferred_element_type=jnp.float32)


def _tc_l2(acc1, hp, dinv, b1, w2p):
    return pl.pallas_call(
        _l2_body,
        grid=(GRID,),
        in_specs=[
            pl.BlockSpec((NC, BLK, HALF), lambda i: (0, i, 0)),
            pl.BlockSpec((NC, BLK, HALF), lambda i: (0, i, 0)),
            pl.BlockSpec((BLK, 1), lambda i: (i, 0)),
            pl.BlockSpec((1, H), lambda i: (0, 0)),
            pl.BlockSpec((H, C_PAD), lambda i: (0, 0)),
        ],
        out_specs=pl.BlockSpec((BLK, C_PAD), lambda i: (i, 0)),
        out_shape=jax.ShapeDtypeStruct((N, C_PAD), jnp.float32),
    )(acc1, hp, dinv, b1, w2p)


def _final_body(acc_ref, h2p_ref, dinv_ref, b2_ref, out_ref):
    a = acc_ref[0] + acc_ref[1] + h2p_ref[...]
    o = (dinv_ref[...] * a)[:, :C] + b2_ref[...]
    m = jnp.max(o, axis=1, keepdims=True)
    z = o - m
    lse = jnp.log(jnp.sum(jnp.exp(z), axis=1, keepdims=True))
    out_ref[...] = z - lse


def _tc_final(acc2, h2p, dinv, b2):
    return pl.pallas_call(
        _final_body,
        grid=(GRID,),
        in_specs=[
            pl.BlockSpec((NC, BLK, C_PAD), lambda i: (0, i, 0)),
            pl.BlockSpec((BLK, C_PAD), lambda i: (i, 0)),
            pl.BlockSpec((BLK, 1), lambda i: (i, 0)),
            pl.BlockSpec((1, C), lambda i: (0, 0)),
        ],
        out_specs=pl.BlockSpec((BLK, C), lambda i: (i, 0)),
        out_shape=jax.ShapeDtypeStruct((N, C), jnp.float32),
    )(acc2, h2p, dinv, b2)


@jax.jit
def kernel(x, edge_index, W1, b1, W2, b2):
    src = edge_index[0]
    dst = edge_index[1]

    ones16 = jnp.ones((CHUNK, 128), jnp.float32)
    zero16 = jnp.zeros((RL, 128), jnp.float32)
    zero_half = jnp.zeros((RL, HALF), jnp.float32)
    zero_cp = jnp.zeros((RL, C_PAD), jnp.float32)
    w2p = jnp.zeros((H, C_PAD), jnp.float32).at[:, :C].set(W2)

    degp = _sc_degree(dst, ones16, zero16)
    dinv = _tc_dinv(degp)
    hp = _tc_l1(x, W1, dinv)
    acc1 = _sc_prop1(hp.reshape(NC * N, HALF), src, dst, zero_half)
    h2p = _tc_l2(acc1, hp, dinv, b1.reshape(1, H), w2p)
    acc2 = _sc_prop2(h2p, src, dst, zero_cp)
    return _tc_final(acc2, h2p, dinv, b2.reshape(1, C))
